# baseline (speedup 1.0000x reference)
import functools

import jax
import jax.numpy as jnp
from jax import lax
from jax.experimental import pallas as pl
from jax.experimental.pallas import tpu as pltpu
from jax.experimental.pallas import tpu_sc as plsc

_EMBED_DIM = 128
_NUM_CODES = 512
_NUM_SUBSPACES = 16
_DS = _EMBED_DIM // _NUM_SUBSPACES
_BETA = 0.25
_PREC = lax.Precision.DEFAULT

_SC_CORES = 2
_SC_SUBCORES = 16
_SC_LANES = 16
_SC_WORKERS = _SC_CORES * _SC_SUBCORES


def _vq_dist_block(cb_ref, cb2_ref, z_ref, idx_ref, loss_ref):
    z = z_ref[0]
    t = z.shape[1]
    n_tiles = _NUM_CODES // _DS
    loss_acc = jnp.zeros((1, 1), jnp.float32)
    sub_f = lax.broadcasted_iota(
        jnp.int32, (_DS, t), 0).astype(jnp.float32)
    big = jnp.float32(_NUM_CODES)
    for n in range(_NUM_SUBSPACES):
        zn = z[n * _DS:(n + 1) * _DS, :]
        cbn = cb_ref[n]
        z_sq = jnp.sum(zn * zn, axis=0, keepdims=True)
        cb_sq = jnp.sum(cbn * cbn, axis=1, keepdims=True)
        inter2 = lax.dot_general(
            cb2_ref[n], zn, (((1,), (0,)), ((), ())),
            precision=_PREC, preferred_element_type=jnp.float32)
        dists = (z_sq + cb_sq) + inter2
        vals = dists[0:_DS]
        tidx = jnp.zeros((_DS, t), jnp.float32)
        for k in range(1, n_tiles):
            dk = dists[k * _DS:(k + 1) * _DS]
            le = vals <= dk
            tidx = jnp.where(le, tidx, jnp.float32(k))
            vals = jnp.minimum(vals, dk)
        dmin = jnp.min(vals, axis=0, keepdims=True)
        cand = tidx * jnp.float32(_DS) + sub_f
        idxf = jnp.min(jnp.where(vals == dmin, cand, big),
                       axis=0, keepdims=True)
        idx_ref[0, n, :] = idxf[0].astype(jnp.int32)
        loss_acc = loss_acc + jnp.sum(dmin, keepdims=True)
    loss_ref[0, :, :] = loss_acc


def _zq_gather_body(cbt_hbm, idx_hbm, out_hbm, idx2_hbm, cbt_vm, idx_vm,
                    stage_vm, idx2_vm, osem0, osem1):
    c = lax.axis_index("c")
    s = lax.axis_index("s")
    wid = s * _SC_CORES + c
    pltpu.sync_copy(cbt_hbm, cbt_vm)
    t = 1024
    blk = _DS * t
    n_chunks = t // _SC_LANES
    st16 = lax.iota(jnp.int32, _SC_LANES) * _NUM_SUBSPACES
    copies = [None, None]
    for rep in range(2):
        b = wid * 2 + rep
        pltpu.sync_copy(idx_hbm.at[b], idx_vm)
        for n in range(_NUM_SUBSPACES):
            buf = n % 2
            sem = osem0 if buf == 0 else osem1
            if copies[buf] is not None:
                copies[buf].wait()
            base = buf * blk
            nbase = n * t

            def chunk(ci, _):
                iv = idx_vm[pl.ds(nbase + ci * _SC_LANES, _SC_LANES)]
                plsc.store_scatter(
                    idx2_vm,
                    [st16 + (ci * _SC_LANES * _NUM_SUBSPACES + n)], iv)
                for d8 in range(_DS):
                    row = plsc.load_gather(
                        cbt_vm, [iv + jnp.int32((n * _DS + d8) * _NUM_CODES)])
                    stage_vm[pl.ds(base + d8 * t + ci * _SC_LANES,
                                   _SC_LANES)] = row
                return 0

            lax.fori_loop(0, n_chunks, chunk, 0, unroll=4)
            copies[buf] = pltpu.async_copy(
                stage_vm.at[pl.ds(base, blk)],
                out_hbm.at[b, pl.ds(n * blk, blk)], sem)
        pltpu.sync_copy(idx2_vm, idx2_hbm.at[b])
    for cp in copies:
        if cp is not None:
            cp.wait()


def _zq_gather(cbt, idx_t, b, d, t):
    mesh = plsc.VectorSubcoreMesh(core_axis_name="c", subcore_axis_name="s")
    fn = pl.kernel(
        _zq_gather_body,
        out_type=(
            jax.ShapeDtypeStruct((b, d * t), jnp.float32),
            jax.ShapeDtypeStruct((b, _NUM_SUBSPACES * t), jnp.int32),
        ),
        mesh=mesh,
        compiler_params=pltpu.CompilerParams(needs_layout_passes=False),
        scratch_types=[
            pltpu.VMEM((d * _NUM_CODES,), jnp.float32),
            pltpu.VMEM((_NUM_SUBSPACES * t,), jnp.int32),
            pltpu.VMEM((2 * _DS * t,), jnp.float32),
            pltpu.VMEM((_NUM_SUBSPACES * t,), jnp.int32),
            pltpu.SemaphoreType.DMA,
            pltpu.SemaphoreType.DMA,
        ],
    )
    return fn(cbt.reshape(-1), idx_t.reshape(b, -1))


def kernel(z, codebooks):
    B, D, H, W = z.shape
    T = H * W
    z3 = z.reshape(B, D, T)
    cb2 = -2.0 * codebooks
    idx_t, loss_parts = pl.pallas_call(
        _vq_dist_block,
        grid=(B,),
        in_specs=[
            pl.BlockSpec((_NUM_SUBSPACES, _NUM_CODES, _DS), lambda i: (0, 0, 0)),
            pl.BlockSpec((_NUM_SUBSPACES, _NUM_CODES, _DS), lambda i: (0, 0, 0)),
            pl.BlockSpec((1, D, T), lambda i: (i, 0, 0)),
        ],
        out_specs=[
            pl.BlockSpec((1, _NUM_SUBSPACES, T), lambda i: (i, 0, 0)),
            pl.BlockSpec((1, 1, 1), lambda i: (i, 0, 0)),
        ],
        out_shape=[
            jax.ShapeDtypeStruct((B, _NUM_SUBSPACES, T), jnp.int32),
            jax.ShapeDtypeStruct((B, 1, 1), jnp.float32),
        ],
    )(codebooks, cb2, z3)
    cbt = jnp.transpose(codebooks, (0, 2, 1)).reshape(D, _NUM_CODES)
    zq_flat, idx2 = _zq_gather(cbt, idx_t, B, D, T)
    z_q = zq_flat.reshape(B, D, H, W)
    indices = idx2.reshape(B, H, W, _NUM_SUBSPACES)
    loss = _BETA * (jnp.sum(loss_parts) / (B * T * D))
    return z_q, loss, indices


# speedup vs baseline: 1.0513x; 1.0513x over previous
import functools

import jax
import jax.numpy as jnp
from jax import lax
from jax.experimental import pallas as pl
from jax.experimental.pallas import tpu as pltpu
from jax.experimental.pallas import tpu_sc as plsc

_EMBED_DIM = 128
_NUM_CODES = 512
_NUM_SUBSPACES = 16
_DS = _EMBED_DIM // _NUM_SUBSPACES
_BETA = 0.25
_PREC = lax.Precision.DEFAULT

_SC_CORES = 2
_SC_SUBCORES = 16
_SC_LANES = 16
_SC_WORKERS = _SC_CORES * _SC_SUBCORES


def _vq_dist_block(cb_ref, cb2_ref, z_ref, idx_ref, loss_ref):
    z = z_ref[0]
    t = z.shape[1]
    n_tiles = _NUM_CODES // _DS
    loss_acc = jnp.zeros((1, 1), jnp.float32)
    sub_f = lax.broadcasted_iota(
        jnp.int32, (_DS, t), 0).astype(jnp.float32)
    big = jnp.float32(_NUM_CODES)
    for n in range(_NUM_SUBSPACES):
        zn = z[n * _DS:(n + 1) * _DS, :]
        cbn = cb_ref[n]
        z_sq = jnp.sum(zn * zn, axis=0, keepdims=True)
        cb_sq = jnp.sum(cbn * cbn, axis=1, keepdims=True)
        inter2 = lax.dot_general(
            cb2_ref[n], zn, (((1,), (0,)), ((), ())),
            precision=_PREC, preferred_element_type=jnp.float32)
        dists = (z_sq + cb_sq) + inter2
        vals = dists[0:_DS]
        tidx = jnp.zeros((_DS, t), jnp.float32)
        for k in range(1, n_tiles):
            dk = dists[k * _DS:(k + 1) * _DS]
            le = vals <= dk
            tidx = jnp.where(le, tidx, jnp.float32(k))
            vals = jnp.minimum(vals, dk)
        dmin = jnp.min(vals, axis=0, keepdims=True)
        cand = tidx * jnp.float32(_DS) + sub_f
        idxf = jnp.min(jnp.where(vals == dmin, cand, big),
                       axis=0, keepdims=True)
        idx_ref[0, n, :] = idxf[0].astype(jnp.int32)
        loss_acc = loss_acc + jnp.sum(dmin, keepdims=True)
    loss_ref[0, :, :] = loss_acc


def _zq_gather_body(cbt_hbm, idx_hbm, out_hbm, idx2_hbm, cbt_vm, idx_vm,
                    stage_vm, idx2_vm, osem0, osem1):
    c = lax.axis_index("c")
    s = lax.axis_index("s")
    wid = s * _SC_CORES + c
    pltpu.sync_copy(cbt_hbm, cbt_vm)
    t = 1024
    blk = _DS * t
    n_chunks = t // _SC_LANES
    st16 = lax.iota(jnp.int32, _SC_LANES) * _NUM_SUBSPACES
    copies = [None, None]
    n_b = idx_hbm.shape[0]
    for rep in range(n_b // _SC_WORKERS):
        b = wid * (n_b // _SC_WORKERS) + rep
        pltpu.sync_copy(idx_hbm.at[b], idx_vm)
        for n in range(_NUM_SUBSPACES):
            buf = n % 2
            sem = osem0 if buf == 0 else osem1
            if copies[buf] is not None:
                copies[buf].wait()
            base = buf * blk
            nbase = n * t

            def chunk(ci, _):
                iv = idx_vm[pl.ds(nbase + ci * _SC_LANES, _SC_LANES)]
                plsc.store_scatter(
                    idx2_vm,
                    [st16 + (ci * _SC_LANES * _NUM_SUBSPACES + n)], iv)
                for d8 in range(_DS):
                    row = plsc.load_gather(
                        cbt_vm, [iv + jnp.int32((n * _DS + d8) * _NUM_CODES)])
                    stage_vm[pl.ds(base + d8 * t + ci * _SC_LANES,
                                   _SC_LANES)] = row
                return 0

            lax.fori_loop(0, n_chunks, chunk, 0, unroll=4)
            copies[buf] = pltpu.async_copy(
                stage_vm.at[pl.ds(base, blk)],
                out_hbm.at[b, pl.ds(n * blk, blk)], sem)
        pltpu.sync_copy(idx2_vm, idx2_hbm.at[b])
    for cp in copies:
        if cp is not None:
            cp.wait()


def _zq_gather(cbt, idx_t, b, d, t):
    mesh = plsc.VectorSubcoreMesh(core_axis_name="c", subcore_axis_name="s")
    fn = pl.kernel(
        _zq_gather_body,
        out_type=(
            jax.ShapeDtypeStruct((b, d * t), jnp.float32),
            jax.ShapeDtypeStruct((b, _NUM_SUBSPACES * t), jnp.int32),
        ),
        mesh=mesh,
        compiler_params=pltpu.CompilerParams(needs_layout_passes=False),
        scratch_types=[
            pltpu.VMEM((d * _NUM_CODES,), jnp.float32),
            pltpu.VMEM((_NUM_SUBSPACES * t,), jnp.int32),
            pltpu.VMEM((2 * _DS * t,), jnp.float32),
            pltpu.VMEM((_NUM_SUBSPACES * t,), jnp.int32),
            pltpu.SemaphoreType.DMA,
            pltpu.SemaphoreType.DMA,
        ],
    )
    return fn(cbt.reshape(-1), idx_t.reshape(b, -1))


def _tc_call(codebooks, cb2, z3, b0, bh, d, t):
    return pl.pallas_call(
        _vq_dist_block,
        grid=(bh,),
        in_specs=[
            pl.BlockSpec((_NUM_SUBSPACES, _NUM_CODES, _DS), lambda i: (0, 0, 0)),
            pl.BlockSpec((_NUM_SUBSPACES, _NUM_CODES, _DS), lambda i: (0, 0, 0)),
            pl.BlockSpec((1, d, t), lambda i: (i + b0, 0, 0)),
        ],
        out_specs=[
            pl.BlockSpec((1, _NUM_SUBSPACES, t), lambda i: (i, 0, 0)),
            pl.BlockSpec((1, 1, 1), lambda i: (i, 0, 0)),
        ],
        out_shape=[
            jax.ShapeDtypeStruct((bh, _NUM_SUBSPACES, t), jnp.int32),
            jax.ShapeDtypeStruct((bh, 1, 1), jnp.float32),
        ],
    )(codebooks, cb2, z3)


def kernel(z, codebooks):
    B, D, H, W = z.shape
    T = H * W
    z3 = z.reshape(B, D, T)
    cb2 = -2.0 * codebooks
    cbt = jnp.transpose(codebooks, (0, 2, 1)).reshape(D, _NUM_CODES)
    nsplit = 2
    bh = B // nsplit
    zq_parts, idx_parts, loss_parts = [], [], []
    for i in range(nsplit):
        idx_t_i, loss_i = _tc_call(codebooks, cb2, z3, i * bh, bh, D, T)
        zq_i, idx2_i = _zq_gather(cbt, idx_t_i, bh, D, T)
        zq_parts.append(zq_i)
        idx_parts.append(idx2_i)
        loss_parts.append(jnp.sum(loss_i))
    z_q = jnp.concatenate(zq_parts, axis=0).reshape(B, D, H, W)
    indices = jnp.concatenate(idx_parts, axis=0).reshape(
        B, H, W, _NUM_SUBSPACES)
    loss = _BETA * (sum(loss_parts) / (B * T * D))
    return z_q, loss, indices


# speedup vs baseline: 1.0526x; 1.0012x over previous
import functools

import jax
import jax.numpy as jnp
from jax import lax
from jax.experimental import pallas as pl
from jax.experimental.pallas import tpu as pltpu
from jax.experimental.pallas import tpu_sc as plsc

_EMBED_DIM = 128
_NUM_CODES = 512
_NUM_SUBSPACES = 16
_DS = _EMBED_DIM // _NUM_SUBSPACES
_BETA = 0.25
_PREC = lax.Precision.DEFAULT

_SC_CORES = 2
_SC_SUBCORES = 16
_SC_LANES = 16
_SC_WORKERS = _SC_CORES * _SC_SUBCORES


def _vq_dist_block(cb_ref, cb2_ref, z_ref, idx_ref, loss_ref):
    z = z_ref[0]
    t = z.shape[1]
    n_tiles = _NUM_CODES // _DS
    loss_acc = jnp.zeros((1, 1), jnp.float32)
    sub_f = lax.broadcasted_iota(
        jnp.int32, (_DS, t), 0).astype(jnp.float32)
    big = jnp.float32(_NUM_CODES)
    for n in range(_NUM_SUBSPACES):
        zn = z[n * _DS:(n + 1) * _DS, :]
        cbn = cb_ref[n]
        z_sq = jnp.sum(zn * zn, axis=0, keepdims=True)
        cb_sq = jnp.sum(cbn * cbn, axis=1, keepdims=True)
        inter2 = lax.dot_general(
            cb2_ref[n], zn, (((1,), (0,)), ((), ())),
            precision=_PREC, preferred_element_type=jnp.float32)
        dists = (z_sq + cb_sq) + inter2
        vals = dists[0:_DS]
        tidx = jnp.zeros((_DS, t), jnp.float32)
        for k in range(1, n_tiles):
            dk = dists[k * _DS:(k + 1) * _DS]
            le = vals <= dk
            tidx = jnp.where(le, tidx, jnp.float32(k))
            vals = jnp.minimum(vals, dk)
        dmin = jnp.min(vals, axis=0, keepdims=True)
        cand = tidx * jnp.float32(_DS) + sub_f
        idxf = jnp.min(jnp.where(vals == dmin, cand, big),
                       axis=0, keepdims=True)
        idx_ref[0, n, :] = idxf[0].astype(jnp.int32)
        loss_acc = loss_acc + jnp.sum(dmin, keepdims=True)
    loss_ref[0, :, :] = loss_acc


def _zq_gather_body(cbt_hbm, idx_hbm, out_hbm, idx2_hbm, cbt_vm, idx_vm,
                    stage_vm, idx2_vm, osem0, osem1):
    c = lax.axis_index("c")
    s = lax.axis_index("s")
    wid = s * _SC_CORES + c
    pltpu.sync_copy(cbt_hbm, cbt_vm)
    t = 1024
    blk = _DS * t
    n_chunks = t // _SC_LANES
    st16 = lax.iota(jnp.int32, _SC_LANES) * _NUM_SUBSPACES
    copies = [None, None]
    n_b = idx_hbm.shape[0]
    for rep in range(n_b // _SC_WORKERS):
        b = wid * (n_b // _SC_WORKERS) + rep
        pltpu.sync_copy(idx_hbm.at[b], idx_vm)
        for n in range(_NUM_SUBSPACES):
            buf = n % 2
            sem = osem0 if buf == 0 else osem1
            if copies[buf] is not None:
                copies[buf].wait()
            base = buf * blk
            nbase = n * t

            def chunk(ci, _):
                iv = idx_vm[pl.ds(nbase + ci * _SC_LANES, _SC_LANES)]
                plsc.store_scatter(
                    idx2_vm,
                    [st16 + (ci * _SC_LANES * _NUM_SUBSPACES + n)], iv)
                for d8 in range(_DS):
                    row = plsc.load_gather(
                        cbt_vm, [iv + jnp.int32((n * _DS + d8) * _NUM_CODES)])
                    stage_vm[pl.ds(base + d8 * t + ci * _SC_LANES,
                                   _SC_LANES)] = row
                return 0

            lax.fori_loop(0, n_chunks, chunk, 0, unroll=8)
            copies[buf] = pltpu.async_copy(
                stage_vm.at[pl.ds(base, blk)],
                out_hbm.at[b, pl.ds(n * blk, blk)], sem)
        pltpu.sync_copy(idx2_vm, idx2_hbm.at[b])
    for cp in copies:
        if cp is not None:
            cp.wait()


def _zq_gather(cbt, idx_t, b, d, t):
    mesh = plsc.VectorSubcoreMesh(core_axis_name="c", subcore_axis_name="s")
    fn = pl.kernel(
        _zq_gather_body,
        out_type=(
            jax.ShapeDtypeStruct((b, d * t), jnp.float32),
            jax.ShapeDtypeStruct((b, _NUM_SUBSPACES * t), jnp.int32),
        ),
        mesh=mesh,
        compiler_params=pltpu.CompilerParams(needs_layout_passes=False),
        scratch_types=[
            pltpu.VMEM((d * _NUM_CODES,), jnp.float32),
            pltpu.VMEM((_NUM_SUBSPACES * t,), jnp.int32),
            pltpu.VMEM((2 * _DS * t,), jnp.float32),
            pltpu.VMEM((_NUM_SUBSPACES * t,), jnp.int32),
            pltpu.SemaphoreType.DMA,
            pltpu.SemaphoreType.DMA,
        ],
    )
    return fn(cbt.reshape(-1), idx_t.reshape(b, -1))


def _tc_call(codebooks, cb2, z3, b0, bh, d, t):
    return pl.pallas_call(
        _vq_dist_block,
        grid=(bh,),
        in_specs=[
            pl.BlockSpec((_NUM_SUBSPACES, _NUM_CODES, _DS), lambda i: (0, 0, 0)),
            pl.BlockSpec((_NUM_SUBSPACES, _NUM_CODES, _DS), lambda i: (0, 0, 0)),
            pl.BlockSpec((1, d, t), lambda i: (i + b0, 0, 0)),
        ],
        out_specs=[
            pl.BlockSpec((1, _NUM_SUBSPACES, t), lambda i: (i, 0, 0)),
            pl.BlockSpec((1, 1, 1), lambda i: (i, 0, 0)),
        ],
        out_shape=[
            jax.ShapeDtypeStruct((bh, _NUM_SUBSPACES, t), jnp.int32),
            jax.ShapeDtypeStruct((bh, 1, 1), jnp.float32),
        ],
    )(codebooks, cb2, z3)


def kernel(z, codebooks):
    B, D, H, W = z.shape
    T = H * W
    z3 = z.reshape(B, D, T)
    cb2 = -2.0 * codebooks
    cbt = jnp.transpose(codebooks, (0, 2, 1)).reshape(D, _NUM_CODES)
    nsplit = 2
    bh = B // nsplit
    zq_parts, idx_parts, loss_parts = [], [], []
    for i in range(nsplit):
        idx_t_i, loss_i = _tc_call(codebooks, cb2, z3, i * bh, bh, D, T)
        zq_i, idx2_i = _zq_gather(cbt, idx_t_i, bh, D, T)
        zq_parts.append(zq_i)
        idx_parts.append(idx2_i)
        loss_parts.append(jnp.sum(loss_i))
    z_q = jnp.concatenate(zq_parts, axis=0).reshape(B, D, H, W)
    indices = jnp.concatenate(idx_parts, axis=0).reshape(
        B, H, W, _NUM_SUBSPACES)
    loss = _BETA * (sum(loss_parts) / (B * T * D))
    return z_q, loss, indices


# speedup vs baseline: 1.0878x; 1.0334x over previous
import functools

import jax
import jax.numpy as jnp
from jax import lax
from jax.experimental import pallas as pl
from jax.experimental.pallas import tpu as pltpu
from jax.experimental.pallas import tpu_sc as plsc

_EMBED_DIM = 128
_NUM_CODES = 512
_NUM_SUBSPACES = 16
_DS = _EMBED_DIM // _NUM_SUBSPACES
_BETA = 0.25
_PREC = lax.Precision.DEFAULT

_SC_CORES = 2
_SC_SUBCORES = 16
_SC_LANES = 16
_SC_WORKERS = _SC_CORES * _SC_SUBCORES


def _vq_dist_block(cb_ref, cb2_ref, z_ref, idx_ref, loss_ref):
    z = z_ref[0]
    t = z.shape[1]
    n_tiles = _NUM_CODES // _DS
    loss_acc = jnp.zeros((1, 1), jnp.float32)
    sub_f = lax.broadcasted_iota(
        jnp.int32, (_DS, t), 0).astype(jnp.float32)
    big = jnp.float32(_NUM_CODES)
    for n in range(_NUM_SUBSPACES):
        zn = z[n * _DS:(n + 1) * _DS, :]
        cbn = cb_ref[n]
        z_sq = jnp.sum(zn * zn, axis=0, keepdims=True)
        cb_sq = jnp.sum(cbn * cbn, axis=1, keepdims=True)
        inter2 = lax.dot_general(
            cb2_ref[n], zn, (((1,), (0,)), ((), ())),
            precision=_PREC, preferred_element_type=jnp.float32)
        dists = (z_sq + cb_sq) + inter2
        vals = dists[0:_DS]
        tidx = jnp.zeros((_DS, t), jnp.float32)
        for k in range(1, n_tiles):
            dk = dists[k * _DS:(k + 1) * _DS]
            le = vals <= dk
            tidx = jnp.where(le, tidx, jnp.float32(k))
            vals = jnp.minimum(vals, dk)
        dmin = jnp.min(vals, axis=0, keepdims=True)
        cand = tidx * jnp.float32(_DS) + sub_f
        idxf = jnp.min(jnp.where(vals == dmin, cand, big),
                       axis=0, keepdims=True)
        idx_ref[0, n, :] = idxf[0].astype(jnp.int32)
        loss_acc = loss_acc + jnp.sum(dmin, keepdims=True)
    loss_ref[0, :, :] = loss_acc


def _zq_gather_body(cbt_hbm, idx_hbm, out_hbm, cbt_vm, idx_vm, stage_vm,
                    osem0, osem1):
    c = lax.axis_index("c")
    s = lax.axis_index("s")
    wid = s * _SC_CORES + c
    pltpu.sync_copy(cbt_hbm, cbt_vm)
    t = 1024
    blk = _DS * t
    half = _NUM_SUBSPACES // 2
    n_chunks = t // _SC_LANES
    copies = [None, None]
    jobs = idx_hbm.shape[0] * 2
    for rep in range(jobs // _SC_WORKERS):
        job = wid * (jobs // _SC_WORKERS) + rep
        b = job // 2
        h = job % 2
        pltpu.sync_copy(idx_hbm.at[b, pl.ds(h * half * t, half * t)], idx_vm)
        for nn in range(half):
            n = h * half + nn
            buf = nn % 2
            sem = osem0 if buf == 0 else osem1
            if copies[buf] is not None:
                copies[buf].wait()
            base = buf * blk
            nbase = nn * t

            def chunk(ci, _):
                iv = idx_vm[pl.ds(nbase + ci * _SC_LANES, _SC_LANES)]
                for d8 in range(_DS):
                    row = plsc.load_gather(
                        cbt_vm, [iv + jnp.int32((n * _DS + d8) * _NUM_CODES)])
                    stage_vm[pl.ds(base + d8 * t + ci * _SC_LANES,
                                   _SC_LANES)] = row
                return 0

            lax.fori_loop(0, n_chunks, chunk, 0, unroll=8)
            copies[buf] = pltpu.async_copy(
                stage_vm.at[pl.ds(base, blk)],
                out_hbm.at[b, pl.ds(n * blk, blk)], sem)
    for cp in copies:
        if cp is not None:
            cp.wait()


def _zq_gather(cbt, idx_t, b, d, t):
    mesh = plsc.VectorSubcoreMesh(core_axis_name="c", subcore_axis_name="s")
    fn = pl.kernel(
        _zq_gather_body,
        out_type=jax.ShapeDtypeStruct((b, d * t), jnp.float32),
        mesh=mesh,
        compiler_params=pltpu.CompilerParams(needs_layout_passes=False),
        scratch_types=[
            pltpu.VMEM((d * _NUM_CODES,), jnp.float32),
            pltpu.VMEM((_NUM_SUBSPACES * t // 2,), jnp.int32),
            pltpu.VMEM((2 * _DS * t,), jnp.float32),
            pltpu.SemaphoreType.DMA,
            pltpu.SemaphoreType.DMA,
        ],
    )
    return fn(cbt.reshape(-1), idx_t.reshape(b, -1))


def _tc_call(codebooks, cb2, z3, b0, bh, d, t):
    return pl.pallas_call(
        _vq_dist_block,
        grid=(bh,),
        in_specs=[
            pl.BlockSpec((_NUM_SUBSPACES, _NUM_CODES, _DS), lambda i: (0, 0, 0)),
            pl.BlockSpec((_NUM_SUBSPACES, _NUM_CODES, _DS), lambda i: (0, 0, 0)),
            pl.BlockSpec((1, d, t), lambda i: (i + b0, 0, 0)),
        ],
        out_specs=[
            pl.BlockSpec((1, _NUM_SUBSPACES, t), lambda i: (i, 0, 0)),
            pl.BlockSpec((1, 1, 1), lambda i: (i, 0, 0)),
        ],
        out_shape=[
            jax.ShapeDtypeStruct((bh, _NUM_SUBSPACES, t), jnp.int32),
            jax.ShapeDtypeStruct((bh, 1, 1), jnp.float32),
        ],
    )(codebooks, cb2, z3)


def kernel(z, codebooks):
    B, D, H, W = z.shape
    T = H * W
    z3 = z.reshape(B, D, T)
    cb2 = -2.0 * codebooks
    cbt = jnp.transpose(codebooks, (0, 2, 1)).reshape(D, _NUM_CODES)
    splits = ((0, 3 * B // 4), (3 * B // 4, B // 4))
    zq_parts, idx_parts, loss_parts = [], [], []
    for b0, bh in splits:
        idx_t_i, loss_i = _tc_call(codebooks, cb2, z3, b0, bh, D, T)
        zq_i = _zq_gather(cbt, idx_t_i, bh, D, T)
        zq_parts.append(zq_i)
        idx_parts.append(idx_t_i)
        loss_parts.append(jnp.sum(loss_i))
    z_q = jnp.concatenate(zq_parts, axis=0).reshape(B, D, H, W)
    idx_t = jnp.concatenate(idx_parts, axis=0)
    indices = jnp.transpose(idx_t, (0, 2, 1)).reshape(B, H, W, _NUM_SUBSPACES)
    loss = _BETA * (sum(loss_parts) / (B * T * D))
    return z_q, loss, indices


# speedup vs baseline: 1.0935x; 1.0053x over previous
import jax
import jax.numpy as jnp
from jax import lax
from jax.experimental import pallas as pl
from jax.experimental.pallas import tpu as pltpu
from jax.experimental.pallas import tpu_sc as plsc

_EMBED_DIM = 128
_NUM_CODES = 512
_NUM_SUBSPACES = 16
_DS = _EMBED_DIM // _NUM_SUBSPACES
_BETA = 0.25
_PREC = lax.Precision.DEFAULT

_SC_CORES = 2
_SC_SUBCORES = 16
_SC_LANES = 16
_SC_WORKERS = _SC_CORES * _SC_SUBCORES


def _vq_dist_block(cb_ref, cb2_ref, z_ref, idx_ref, loss_ref):
    z = z_ref[0]
    t = z.shape[1]
    n_tiles = _NUM_CODES // _DS
    loss_acc = jnp.zeros((1, 1), jnp.float32)
    sub_f = lax.broadcasted_iota(
        jnp.int32, (_DS, t), 0).astype(jnp.float32)
    big = jnp.float32(_NUM_CODES)
    for n in range(_NUM_SUBSPACES):
        zn = z[n * _DS:(n + 1) * _DS, :]
        cbn = cb_ref[n]
        z_sq = jnp.sum(zn * zn, axis=0, keepdims=True)
        cb_sq = jnp.sum(cbn * cbn, axis=1, keepdims=True)
        inter2 = lax.dot_general(
            cb2_ref[n], zn, (((1,), (0,)), ((), ())),
            precision=_PREC, preferred_element_type=jnp.float32)
        dists = (z_sq + cb_sq) + inter2
        vals = dists[0:_DS]
        tidx = jnp.zeros((_DS, t), jnp.float32)
        for k in range(1, n_tiles):
            dk = dists[k * _DS:(k + 1) * _DS]
            le = vals <= dk
            tidx = jnp.where(le, tidx, jnp.float32(k))
            vals = jnp.minimum(vals, dk)
        dmin = jnp.min(vals, axis=0, keepdims=True)
        cand = tidx * jnp.float32(_DS) + sub_f
        idxf = jnp.min(jnp.where(vals == dmin, cand, big),
                       axis=0, keepdims=True)
        idx_ref[0, n, :] = idxf[0].astype(jnp.int32)
        loss_acc = loss_acc + jnp.sum(dmin, keepdims=True)
    loss_ref[0, :, :] = loss_acc


def _zq_gather_body(cbt_hbm, idx_hbm, out_hbm, cbt_vm, idx_vm, stage_vm,
                    osem0, osem1):
    c = lax.axis_index("c")
    s = lax.axis_index("s")
    wid = s * _SC_CORES + c
    pltpu.sync_copy(cbt_hbm, cbt_vm)
    t = 1024
    blk = _DS * t
    half = _NUM_SUBSPACES // 2
    n_chunks = t // _SC_LANES
    copies = [None, None]
    jobs = idx_hbm.shape[0] * 2
    for rep in range(jobs // _SC_WORKERS):
        job = wid * (jobs // _SC_WORKERS) + rep
        b = job // 2
        h = job % 2
        pltpu.sync_copy(idx_hbm.at[b, pl.ds(h * half * t, half * t)], idx_vm)
        for nn in range(half):
            n = h * half + nn
            buf = nn % 2
            sem = osem0 if buf == 0 else osem1
            if copies[buf] is not None:
                copies[buf].wait()
            base = buf * blk
            nbase = nn * t

            def chunk(ci, _):
                iv = idx_vm[pl.ds(nbase + ci * _SC_LANES, _SC_LANES)]
                for d8 in range(_DS):
                    row = plsc.load_gather(
                        cbt_vm, [iv + jnp.int32((n * _DS + d8) * _NUM_CODES)])
                    stage_vm[pl.ds(base + d8 * t + ci * _SC_LANES,
                                   _SC_LANES)] = row
                return 0

            lax.fori_loop(0, n_chunks, chunk, 0, unroll=8)
            copies[buf] = pltpu.async_copy(
                stage_vm.at[pl.ds(base, blk)],
                out_hbm.at[b, pl.ds(n * blk, blk)], sem)
    for cp in copies:
        if cp is not None:
            cp.wait()


def _zq_gather(cbt, idx_t, b, d, t):
    mesh = plsc.VectorSubcoreMesh(core_axis_name="c", subcore_axis_name="s")
    fn = pl.kernel(
        _zq_gather_body,
        out_type=jax.ShapeDtypeStruct((b, d * t), jnp.float32),
        mesh=mesh,
        compiler_params=pltpu.CompilerParams(needs_layout_passes=False),
        scratch_types=[
            pltpu.VMEM((d * _NUM_CODES,), jnp.float32),
            pltpu.VMEM((_NUM_SUBSPACES * t // 2,), jnp.int32),
            pltpu.VMEM((2 * _DS * t,), jnp.float32),
            pltpu.SemaphoreType.DMA,
            pltpu.SemaphoreType.DMA,
        ],
    )
    return fn(cbt.reshape(-1), idx_t.reshape(b, -1))


def _tc_call(codebooks, cb2, z3, b0, bh, d, t):
    return pl.pallas_call(
        _vq_dist_block,
        grid=(bh,),
        in_specs=[
            pl.BlockSpec((_NUM_SUBSPACES, _NUM_CODES, _DS), lambda i: (0, 0, 0)),
            pl.BlockSpec((_NUM_SUBSPACES, _NUM_CODES, _DS), lambda i: (0, 0, 0)),
            pl.BlockSpec((1, d, t), lambda i: (i + b0, 0, 0)),
        ],
        out_specs=[
            pl.BlockSpec((1, _NUM_SUBSPACES, t), lambda i: (i, 0, 0)),
            pl.BlockSpec((1, 1, 1), lambda i: (i, 0, 0)),
        ],
        out_shape=[
            jax.ShapeDtypeStruct((bh, _NUM_SUBSPACES, t), jnp.int32),
            jax.ShapeDtypeStruct((bh, 1, 1), jnp.float32),
        ],
    )(codebooks, cb2, z3)


def kernel(z, codebooks):
    B, D, H, W = z.shape
    T = H * W
    z3 = z.reshape(B, D, T)
    cb2 = -2.0 * codebooks
    cbt = jnp.transpose(codebooks, (0, 2, 1)).reshape(D, _NUM_CODES)
    splits = ((0, 3 * B // 4), (3 * B // 4, B // 4))
    zq_parts, idx_parts, loss_parts = [], [], []
    for b0, bh in splits:
        idx_t_i, loss_i = _tc_call(codebooks, cb2, z3, b0, bh, D, T)
        zq_i = _zq_gather(cbt, idx_t_i, bh, D, T)
        zq_parts.append(zq_i)
        idx_parts.append(idx_t_i)
        loss_parts.append(jnp.sum(loss_i))
    z_q = jnp.concatenate(zq_parts, axis=0).reshape(B, D, H, W)
    idx_t = jnp.concatenate(idx_parts, axis=0)
    indices = jnp.transpose(idx_t, (0, 2, 1)).reshape(B, H, W, _NUM_SUBSPACES)
    loss = _BETA * (sum(loss_parts) / (B * T * D))
    return z_q, loss, indices
